# baseline (speedup 1.0000x reference)
import functools

import jax
import jax.numpy as jnp
from jax import lax
from jax.experimental import pallas as pl
from jax.experimental.pallas import tpu as pltpu
from jax.experimental.pallas import tpu_sc as plsc

_F32 = jnp.float32
_BR = 512
_CH = 128
_NW = 32
_G = 64
_SELU_L = 1.0507009873554805
_SELU_A = 1.6732632423543772


def _selu(x):
    return _SELU_L * jnp.where(x > 0, x, _SELU_A * (jnp.exp(jnp.minimum(x, 0.0)) - 1.0))


def _make_sc_agg(NP, D, nchunks):
    mesh = plsc.VectorSubcoreMesh(core_axis_name="c", subcore_axis_name="s")
    rows_per = NP // 16
    Dh = D // 2

    def body(hlo_hbm, hhi_hbm, src_hbm, dst_hbm, out_hbm,
             idx_s, idx_d, r0, r1, r2, r3, acc_sh,
             sg0, sg1, sg2, sg3, ss0, ss1, ss2, ss3):
        rows = [r0, r1, r2, r3]
        sem_g = [sg0, sg1, sg2, sg3]
        sem_s = [ss0, ss1, ss2, ss3]
        c = lax.axis_index("c")
        s = lax.axis_index("s")
        pltpu.sync_copy(src_hbm.at[s], idx_s)
        pltpu.sync_copy(dst_hbm.at[s], idx_d)

        nbuf = len(rows)

        def pipeline(h_hbm):
            pltpu.sync_copy(h_hbm.at[pl.ds(s * rows_per, rows_per)],
                            acc_sh.at[pl.ds(s * rows_per, rows_per)])
            plsc.subcore_barrier()
            for b in range(nbuf):
                pltpu.async_copy(h_hbm.at[idx_s.at[b]], rows[b], sem_g[b])

            def step(k, carry):
                base = k * nbuf
                for b in range(nbuf):
                    ci = base + b
                    pltpu.make_async_copy(h_hbm.at[idx_s.at[ci]], rows[b],
                                          sem_g[b]).wait()
                    pltpu.async_copy(rows[b], acc_sh.at[idx_d.at[ci]],
                                     sem_s[b], add=True)
                for b in range(nbuf):
                    ci = base + nbuf + b

                    @pl.when(ci < nchunks)
                    def _(ci=ci, b=b):
                        pltpu.make_async_copy(rows[b],
                                              acc_sh.at[idx_d.at[base + b]],
                                              sem_s[b]).wait()
                        pltpu.async_copy(h_hbm.at[idx_s.at[ci]], rows[b],
                                         sem_g[b])
                return carry

            lax.fori_loop(0, nchunks // nbuf, step, 0)
            for b in range(nbuf):
                pltpu.make_async_copy(rows[b], acc_sh.at[idx_d.at[0]],
                                      sem_s[b]).wait()

        @pl.when(c == 0)
        def _():
            pipeline(hlo_hbm)

        @pl.when(c == 1)
        def _():
            pipeline(hhi_hbm)

        plsc.subcore_barrier()
        pltpu.sync_copy(acc_sh.at[pl.ds(s * rows_per, rows_per)],
                        out_hbm.at[c, pl.ds(s * rows_per, rows_per)])

    return pl.kernel(
        body,
        out_type=jax.ShapeDtypeStruct((2, NP, Dh), _F32),
        mesh=mesh,
        compiler_params=pltpu.CompilerParams(use_tc_tiling_on_sc=False),
        scratch_types=(
            [pltpu.VMEM((nchunks, _CH), jnp.int32),
             pltpu.VMEM((nchunks, _CH), jnp.int32)]
            + [pltpu.VMEM((_CH, Dh), _F32)] * 4
            + [pltpu.VMEM_SHARED((NP, Dh), _F32)]
            + [pltpu.SemaphoreType.DMA] * 8
        ),
    )


def _onehot(b_ref):
    bcol = b_ref[:, 0:1]
    gid = lax.broadcasted_iota(jnp.int32, (1, _G), 1).astype(_F32)
    return jnp.where(bcol == gid, 1.0, 0.0).astype(_F32)


def _make_fused(NP, D, H, nreal):
    NB = NP // _BR
    Dh = D // 2
    HIGH = lax.Precision.HIGHEST

    def body(a2_ref, b_ref, w1_ref, g1_ref, b1_ref, w2_ref, b2_ref,
             gnw_ref, gnb_ref, gna_ref, hlo_ref, hhi_ref, pool_ref,
             t_s, c_s, s_s, ab_s, stst_s, cnt8_s, mean_s, rstd_s, cntb_s):
        p = pl.program_id(0)
        i = pl.program_id(1)

        @pl.when(p == 0)
        def _():
            h2 = jnp.concatenate([a2_ref[0], a2_ref[1]], axis=1)
            rid = lax.broadcasted_iota(jnp.int32, (_BR, 1), 0) + i * _BR
            h2 = jnp.where(rid < nreal, h2, 0.0)
            h2c = h2.astype(jnp.bfloat16).astype(_F32)
            cb = lax.dot_general(h2c, h2c, (((0,), (0,)), ((), ())),
                                 preferred_element_type=_F32, precision=HIGH)
            sb = jnp.broadcast_to(jnp.sum(h2c, axis=0, keepdims=True), (8, D))

            @pl.when(i == 0)
            def _():
                c_s[...] = cb
                s_s[...] = sb

            @pl.when(i != 0)
            def _():
                c_s[...] += cb
                s_s[...] += sb

        @pl.when(p == 1)
        def _():
            @pl.when(i == 0)
            def _():
                w1 = w1_ref[...].astype(jnp.bfloat16).astype(_F32)
                mu = lax.dot_general(s_s[0:1, :], w1, (((1,), (1,)), ((), ())),
                                     preferred_element_type=_F32,
                                     precision=HIGH) / nreal
                wc = lax.dot_general(w1, c_s[...], (((1,), (0,)), ((), ())),
                                     preferred_element_type=_F32, precision=HIGH)
                m2col = jnp.sum(wc * w1, axis=1, keepdims=True) / nreal
                ii = lax.broadcasted_iota(jnp.int32, (H, H), 0)
                jj = lax.broadcasted_iota(jnp.int32, (H, H), 1)
                eye = jnp.where(ii == jj, 1.0, 0.0).astype(_F32)
                m2 = lax.dot_general(m2col, eye, (((0,), (0,)), ((), ())),
                                     preferred_element_type=_F32,
                                     precision=HIGH)
                rstd = lax.rsqrt(m2 - mu * mu + 1e-5)
                alpha = g1_ref[...] * rstd
                ab_s[0:1, :] = alpha
                ab_s[1:2, :] = b1_ref[...] - mu * alpha

            h2 = jnp.concatenate([a2_ref[0], a2_ref[1]], axis=1)
            y = lax.dot_general(h2.astype(jnp.bfloat16),
                                w1_ref[...].astype(jnp.bfloat16),
                                (((1,), (1,)), ((), ())),
                                preferred_element_type=_F32)
            z = _selu(y * ab_s[0:1, :] + ab_s[1:2, :])
            t = lax.dot_general(z.astype(jnp.bfloat16),
                                w2_ref[...].astype(jnp.bfloat16),
                                (((1,), (1,)), ((), ())),
                                preferred_element_type=_F32) + b2_ref[...]
            t_s[pl.ds(i * _BR, _BR), :] = t
            m = _onehot(b_ref)
            ts = jnp.concatenate([t, t * t], axis=1)
            stb = lax.dot_general(m, ts, (((0,), (0,)), ((), ())),
                                  preferred_element_type=_F32, precision=HIGH)
            cb = jnp.broadcast_to(jnp.sum(m, axis=0, keepdims=True), (8, _G))

            @pl.when(i == 0)
            def _():
                stst_s[...] = stb
                cnt8_s[...] = cb

            @pl.when(i != 0)
            def _():
                stst_s[...] += stb
                cnt8_s[...] += cb

        @pl.when(p == 2)
        def _():
            @pl.when(i == 0)
            def _():
                ii = lax.broadcasted_iota(jnp.int32, (_G, _G), 0)
                jj = lax.broadcasted_iota(jnp.int32, (_G, _G), 1)
                eye = jnp.where(ii == jj, 1.0, 0.0).astype(_F32)
                cntcol = lax.dot_general(eye, cnt8_s[0:1, :],
                                         (((1,), (1,)), ((), ())),
                                         preferred_element_type=_F32,
                                         precision=HIGH)
                cnt = jnp.maximum(cntcol, 1.0)
                mean = stst_s[:, :D] / cnt
                a = gna_ref[...]
                var = stst_s[:, D:] / cnt - (2.0 * a - a * a) * mean * mean
                mean_s[...] = mean
                rstd_s[...] = lax.rsqrt(var + 1e-5)
                cntb_s[...] = jnp.broadcast_to(cnt, (_G, D))
                pool_ref[...] = jnp.zeros((_G, D), _F32)

            t = t_s[pl.ds(i * _BR, _BR), :]
            m = _onehot(b_ref)
            meanb = lax.dot_general(m, mean_s[...], (((1,), (0,)), ((), ())),
                                    preferred_element_type=_F32, precision=HIGH)
            rstdb = lax.dot_general(m, rstd_s[...], (((1,), (0,)), ((), ())),
                                    preferred_element_type=_F32, precision=HIGH)
            out = (t - gna_ref[...] * meanb) * rstdb * gnw_ref[...] \
                + gnb_ref[...]
            hn = _selu(out)
            hlo_ref[...] = hn[:, :Dh]
            hhi_ref[...] = hn[:, Dh:]
            pool_ref[...] += lax.dot_general(m, hn, (((0,), (0,)), ((), ())),
                                             preferred_element_type=_F32,
                                             precision=HIGH) / cntb_s[...]

    return pl.pallas_call(
        body,
        grid=(3, NB),
        in_specs=[pl.BlockSpec((2, _BR, Dh), lambda p, i: (0, i, 0)),
                  pl.BlockSpec((_BR, D), lambda p, i: (i, 0)),
                  pl.BlockSpec((H, D), lambda p, i: (0, 0)),
                  pl.BlockSpec((1, H), lambda p, i: (0, 0)),
                  pl.BlockSpec((1, H), lambda p, i: (0, 0)),
                  pl.BlockSpec((D, H), lambda p, i: (0, 0)),
                  pl.BlockSpec((1, D), lambda p, i: (0, 0)),
                  pl.BlockSpec((1, D), lambda p, i: (0, 0)),
                  pl.BlockSpec((1, D), lambda p, i: (0, 0)),
                  pl.BlockSpec((1, D), lambda p, i: (0, 0))],
        out_specs=[pl.BlockSpec((_BR, Dh), lambda p, i: (i, 0)),
                   pl.BlockSpec((_BR, Dh), lambda p, i: (i, 0)),
                   pl.BlockSpec((_G, D), lambda p, i: (0, 0))],
        out_shape=[jax.ShapeDtypeStruct((NP, Dh), _F32),
                   jax.ShapeDtypeStruct((NP, Dh), _F32),
                   jax.ShapeDtypeStruct((_G, D), _F32)],
        scratch_shapes=[pltpu.VMEM((NP, D), _F32),
                        pltpu.VMEM((D, D), _F32),
                        pltpu.VMEM((8, D), _F32),
                        pltpu.VMEM((2, H), _F32),
                        pltpu.VMEM((_G, 2 * D), _F32),
                        pltpu.VMEM((8, _G), _F32),
                        pltpu.VMEM((_G, D), _F32),
                        pltpu.VMEM((_G, D), _F32),
                        pltpu.VMEM((_G, D), _F32)],
    )


def kernel(x, edge_index, batch,
           W1_1, g1_1, b1_1, W2_1, b2_1, gnw_1, gnb_1, gna_1,
           W1_2, g1_2, b1_2, W2_2, b2_2, gnw_2, gnb_2, gna_2,
           W1_3, g1_3, b1_3, W2_3, b2_3, gnw_3, gnb_3, gna_3,
           W1_4, g1_4, b1_4, W2_4, b2_4, gnw_4, gnb_4, gna_4):
    params = (W1_1, g1_1, b1_1, W2_1, b2_1, gnw_1, gnb_1, gna_1,
              W1_2, g1_2, b1_2, W2_2, b2_2, gnw_2, gnb_2, gna_2,
              W1_3, g1_3, b1_3, W2_3, b2_3, gnw_3, gnb_3, gna_3,
              W1_4, g1_4, b1_4, W2_4, b2_4, gnw_4, gnb_4, gna_4)
    N, D = x.shape
    E = edge_index.shape[1]
    H = W1_1.shape[0]
    Dh = D // 2
    NP = -(-N // _BR) * _BR
    nsub = 16
    nchunks = -(-E // (nsub * _CH))
    nchunks = -(-nchunks // 4) * 4
    EP = nsub * nchunks * _CH
    padr = NP - N

    pad_idx = (jnp.arange(EP - E, dtype=jnp.int32) % padr) + N
    src_r = jnp.concatenate([edge_index[0], pad_idx]).reshape(nsub, nchunks, _CH)
    dst_r = jnp.concatenate([edge_index[1], pad_idx]).reshape(nsub, nchunks, _CH)
    hlo = jnp.pad(x[:, :Dh], ((0, NP - N), (0, 0)))
    hhi = jnp.pad(x[:, Dh:], ((0, NP - N), (0, 0)))
    bp = jnp.pad(batch, (0, NP - N), constant_values=_G)
    bf = jnp.broadcast_to(bp.astype(_F32)[:, None], (NP, D))

    sc_agg = _make_sc_agg(NP, D, nchunks)
    fused = _make_fused(NP, D, H, N)

    pools = []
    for li in range(4):
        w1, g1, b1, w2, b2, gnw, gnb, gna = params[8 * li: 8 * li + 8]
        h2s = sc_agg(hlo, hhi, src_r, dst_r)
        hlo, hhi, pool = fused(h2s, bf, w1,
                               g1.reshape(1, H), b1.reshape(1, H),
                               w2, b2.reshape(1, D),
                               gnw.reshape(1, D), gnb.reshape(1, D),
                               gna.reshape(1, D))
        pools.append(pool)
    return jnp.concatenate(pools, axis=1)


# speedup vs baseline: 1.0113x; 1.0113x over previous
import functools

import jax
import jax.numpy as jnp
from jax import lax
from jax.experimental import pallas as pl
from jax.experimental.pallas import tpu as pltpu
from jax.experimental.pallas import tpu_sc as plsc

_F32 = jnp.float32
_BR = 512
_CH = 128
_NW = 32
_G = 64
_SELU_L = 1.0507009873554805
_SELU_A = 1.6732632423543772


def _selu(x):
    return _SELU_L * jnp.where(x > 0, x, _SELU_A * (jnp.exp(jnp.minimum(x, 0.0)) - 1.0))


def _make_sc_agg(NP, D, nchunks):
    mesh = plsc.VectorSubcoreMesh(core_axis_name="c", subcore_axis_name="s")
    rows_per = NP // 16
    Dh = D // 2

    def body(hlo_hbm, hhi_hbm, src_hbm, dst_hbm, out_hbm,
             idx_s, idx_d, r0, r1, r2, r3, acc_sh,
             sg0, sg1, sg2, sg3, ss0, ss1, ss2, ss3):
        rows = [r0, r1, r2, r3]
        sem_g = [sg0, sg1, sg2, sg3]
        sem_s = [ss0, ss1, ss2, ss3]
        c = lax.axis_index("c")
        s = lax.axis_index("s")
        pltpu.sync_copy(src_hbm.at[s], idx_s)
        pltpu.sync_copy(dst_hbm.at[s], idx_d)

        nbuf = len(rows)

        def pipeline(h_hbm):
            pltpu.sync_copy(h_hbm.at[pl.ds(s * rows_per, rows_per)],
                            acc_sh.at[pl.ds(s * rows_per, rows_per)])
            plsc.subcore_barrier()
            for b in range(nbuf):
                pltpu.async_copy(h_hbm.at[idx_s.at[b]], rows[b], sem_g[b])

            def step(k, carry):
                base = k * nbuf
                for b in range(nbuf):
                    ci = base + b
                    pltpu.make_async_copy(h_hbm.at[idx_s.at[ci]], rows[b],
                                          sem_g[b]).wait()
                    pltpu.async_copy(rows[b], acc_sh.at[idx_d.at[ci]],
                                     sem_s[b], add=True)
                for b in range(nbuf):
                    ci = base + nbuf + b

                    @pl.when(ci < nchunks)
                    def _(ci=ci, b=b):
                        pltpu.make_async_copy(rows[b],
                                              acc_sh.at[idx_d.at[base + b]],
                                              sem_s[b]).wait()
                        pltpu.async_copy(h_hbm.at[idx_s.at[ci]], rows[b],
                                         sem_g[b])
                return carry

            lax.fori_loop(0, nchunks // nbuf, step, 0)
            for b in range(nbuf):
                pltpu.make_async_copy(rows[b], acc_sh.at[idx_d.at[0]],
                                      sem_s[b]).wait()

        @pl.when(c == 0)
        def _():
            pipeline(hlo_hbm)

        @pl.when(c == 1)
        def _():
            pipeline(hhi_hbm)

        plsc.subcore_barrier()
        pltpu.sync_copy(acc_sh.at[pl.ds(s * rows_per, rows_per)],
                        out_hbm.at[c, pl.ds(s * rows_per, rows_per)])

    return pl.kernel(
        body,
        out_type=jax.ShapeDtypeStruct((2, NP, Dh), _F32),
        mesh=mesh,
        compiler_params=pltpu.CompilerParams(use_tc_tiling_on_sc=False),
        scratch_types=(
            [pltpu.VMEM((nchunks, _CH), jnp.int32),
             pltpu.VMEM((nchunks, _CH), jnp.int32)]
            + [pltpu.VMEM((_CH, Dh), _F32)] * 4
            + [pltpu.VMEM_SHARED((NP, Dh), _F32)]
            + [pltpu.SemaphoreType.DMA] * 8
        ),
    )


def _onehot(b_ref):
    bcol = b_ref[:, 0:1]
    gid = lax.broadcasted_iota(jnp.int32, (1, _G), 1).astype(_F32)
    return jnp.where(bcol == gid, 1.0, 0.0).astype(_F32)


def _make_fused(NP, D, H, nreal):
    NB = NP // _BR
    Dh = D // 2
    HIGH = lax.Precision.HIGHEST

    def body(a2_ref, b_ref, w1_ref, g1_ref, b1_ref, w2_ref, b2_ref,
             gnw_ref, gnb_ref, gna_ref, hlo_ref, hhi_ref, pool_ref,
             t_s, c_s, s_s, ab_s, stst_s, cnt8_s, mean_s, rstd_s, cntb_s):
        p = pl.program_id(0)
        i = pl.program_id(1)

        @pl.when(p == 0)
        def _():
            h2 = jnp.concatenate([a2_ref[0], a2_ref[1]], axis=1)
            rid = lax.broadcasted_iota(jnp.int32, (_BR, 1), 0) + i * _BR
            h2 = jnp.where(rid < nreal, h2, 0.0)
            h2c = h2.astype(jnp.bfloat16).astype(_F32)
            cb = lax.dot_general(h2c, h2c, (((0,), (0,)), ((), ())),
                                 preferred_element_type=_F32, precision=HIGH)
            sb = jnp.broadcast_to(jnp.sum(h2c, axis=0, keepdims=True), (8, D))

            @pl.when(i == 0)
            def _():
                c_s[...] = cb
                s_s[...] = sb

            @pl.when(i != 0)
            def _():
                c_s[...] += cb
                s_s[...] += sb

        @pl.when(p == 1)
        def _():
            @pl.when(i == 0)
            def _():
                w1 = w1_ref[...].astype(jnp.bfloat16).astype(_F32)
                mu = lax.dot_general(s_s[0:1, :], w1, (((1,), (1,)), ((), ())),
                                     preferred_element_type=_F32,
                                     precision=HIGH) / nreal
                wc = lax.dot_general(w1, c_s[...], (((1,), (0,)), ((), ())),
                                     preferred_element_type=_F32, precision=HIGH)
                m2col = jnp.sum(wc * w1, axis=1, keepdims=True) / nreal
                ii = lax.broadcasted_iota(jnp.int32, (H, H), 0)
                jj = lax.broadcasted_iota(jnp.int32, (H, H), 1)
                eye = jnp.where(ii == jj, 1.0, 0.0).astype(_F32)
                m2 = lax.dot_general(m2col, eye, (((0,), (0,)), ((), ())),
                                     preferred_element_type=_F32,
                                     precision=HIGH)
                rstd = lax.rsqrt(m2 - mu * mu + 1e-5)
                alpha = g1_ref[...] * rstd
                ab_s[0:1, :] = alpha
                ab_s[1:2, :] = b1_ref[...] - mu * alpha

            h2 = jnp.concatenate([a2_ref[0], a2_ref[1]], axis=1)
            y = lax.dot_general(h2.astype(jnp.bfloat16),
                                w1_ref[...].astype(jnp.bfloat16),
                                (((1,), (1,)), ((), ())),
                                preferred_element_type=_F32)
            z = _selu(y * ab_s[0:1, :] + ab_s[1:2, :])
            t = lax.dot_general(z.astype(jnp.bfloat16),
                                w2_ref[...].astype(jnp.bfloat16),
                                (((1,), (1,)), ((), ())),
                                preferred_element_type=_F32) + b2_ref[...]
            t_s[pl.ds(i * _BR, _BR), :] = t
            m = _onehot(b_ref)
            ts = jnp.concatenate([t, t * t], axis=1)
            stb = lax.dot_general(m, ts, (((0,), (0,)), ((), ())),
                                  preferred_element_type=_F32, precision=HIGH)
            cb = jnp.broadcast_to(jnp.sum(m, axis=0, keepdims=True), (8, _G))

            @pl.when(i == 0)
            def _():
                stst_s[...] = stb
                cnt8_s[...] = cb

            @pl.when(i != 0)
            def _():
                stst_s[...] += stb
                cnt8_s[...] += cb

        @pl.when(p == 2)
        def _():
            @pl.when(i == 0)
            def _():
                ii = lax.broadcasted_iota(jnp.int32, (_G, _G), 0)
                jj = lax.broadcasted_iota(jnp.int32, (_G, _G), 1)
                eye = jnp.where(ii == jj, 1.0, 0.0).astype(_F32)
                cntcol = lax.dot_general(eye, cnt8_s[0:1, :],
                                         (((1,), (1,)), ((), ())),
                                         preferred_element_type=_F32,
                                         precision=HIGH)
                cnt = jnp.maximum(cntcol, 1.0)
                mean = stst_s[:, :D] / cnt
                a = gna_ref[...]
                var = stst_s[:, D:] / cnt - (2.0 * a - a * a) * mean * mean
                mean_s[...] = mean
                rstd_s[...] = lax.rsqrt(var + 1e-5)
                cntb_s[...] = jnp.broadcast_to(cnt, (_G, D))
                pool_ref[...] = jnp.zeros((_G, D), _F32)

            t = t_s[pl.ds(i * _BR, _BR), :]
            m = _onehot(b_ref)
            meanb = lax.dot_general(m, mean_s[...], (((1,), (0,)), ((), ())),
                                    preferred_element_type=_F32, precision=HIGH)
            rstdb = lax.dot_general(m, rstd_s[...], (((1,), (0,)), ((), ())),
                                    preferred_element_type=_F32, precision=HIGH)
            out = (t - gna_ref[...] * meanb) * rstdb * gnw_ref[...] \
                + gnb_ref[...]
            hn = _selu(out)
            hlo_ref[...] = hn[:, :Dh]
            hhi_ref[...] = hn[:, Dh:]
            pool_ref[...] += lax.dot_general(m, hn, (((0,), (0,)), ((), ())),
                                             preferred_element_type=_F32,
                                             precision=HIGH) / cntb_s[...]

    return pl.pallas_call(
        body,
        grid=(3, NB),
        in_specs=[pl.BlockSpec((2, _BR, Dh), lambda p, i: (0, i * (p < 2), 0)),
                  pl.BlockSpec((_BR, D), lambda p, i: (i * (p > 0), 0)),
                  pl.BlockSpec((H, D), lambda p, i: (0, 0)),
                  pl.BlockSpec((1, H), lambda p, i: (0, 0)),
                  pl.BlockSpec((1, H), lambda p, i: (0, 0)),
                  pl.BlockSpec((D, H), lambda p, i: (0, 0)),
                  pl.BlockSpec((1, D), lambda p, i: (0, 0)),
                  pl.BlockSpec((1, D), lambda p, i: (0, 0)),
                  pl.BlockSpec((1, D), lambda p, i: (0, 0)),
                  pl.BlockSpec((1, D), lambda p, i: (0, 0))],
        out_specs=[pl.BlockSpec((_BR, Dh), lambda p, i: (i, 0)),
                   pl.BlockSpec((_BR, Dh), lambda p, i: (i, 0)),
                   pl.BlockSpec((_G, D), lambda p, i: (0, 0))],
        out_shape=[jax.ShapeDtypeStruct((NP, Dh), _F32),
                   jax.ShapeDtypeStruct((NP, Dh), _F32),
                   jax.ShapeDtypeStruct((_G, D), _F32)],
        scratch_shapes=[pltpu.VMEM((NP, D), _F32),
                        pltpu.VMEM((D, D), _F32),
                        pltpu.VMEM((8, D), _F32),
                        pltpu.VMEM((2, H), _F32),
                        pltpu.VMEM((_G, 2 * D), _F32),
                        pltpu.VMEM((8, _G), _F32),
                        pltpu.VMEM((_G, D), _F32),
                        pltpu.VMEM((_G, D), _F32),
                        pltpu.VMEM((_G, D), _F32)],
    )


def kernel(x, edge_index, batch,
           W1_1, g1_1, b1_1, W2_1, b2_1, gnw_1, gnb_1, gna_1,
           W1_2, g1_2, b1_2, W2_2, b2_2, gnw_2, gnb_2, gna_2,
           W1_3, g1_3, b1_3, W2_3, b2_3, gnw_3, gnb_3, gna_3,
           W1_4, g1_4, b1_4, W2_4, b2_4, gnw_4, gnb_4, gna_4):
    params = (W1_1, g1_1, b1_1, W2_1, b2_1, gnw_1, gnb_1, gna_1,
              W1_2, g1_2, b1_2, W2_2, b2_2, gnw_2, gnb_2, gna_2,
              W1_3, g1_3, b1_3, W2_3, b2_3, gnw_3, gnb_3, gna_3,
              W1_4, g1_4, b1_4, W2_4, b2_4, gnw_4, gnb_4, gna_4)
    N, D = x.shape
    E = edge_index.shape[1]
    H = W1_1.shape[0]
    Dh = D // 2
    NP = -(-N // _BR) * _BR
    nsub = 16
    nchunks = -(-E // (nsub * _CH))
    nchunks = -(-nchunks // 4) * 4
    EP = nsub * nchunks * _CH
    padr = NP - N

    pad_idx = (jnp.arange(EP - E, dtype=jnp.int32) % padr) + N
    src_r = jnp.concatenate([edge_index[0], pad_idx]).reshape(nsub, nchunks, _CH)
    dst_r = jnp.concatenate([edge_index[1], pad_idx]).reshape(nsub, nchunks, _CH)
    hlo = jnp.pad(x[:, :Dh], ((0, NP - N), (0, 0)))
    hhi = jnp.pad(x[:, Dh:], ((0, NP - N), (0, 0)))
    bp = jnp.pad(batch, (0, NP - N), constant_values=_G)
    bf = jnp.broadcast_to(bp.astype(_F32)[:, None], (NP, D))

    sc_agg = _make_sc_agg(NP, D, nchunks)
    fused = _make_fused(NP, D, H, N)

    pools = []
    for li in range(4):
        w1, g1, b1, w2, b2, gnw, gnb, gna = params[8 * li: 8 * li + 8]
        h2s = sc_agg(hlo, hhi, src_r, dst_r)
        hlo, hhi, pool = fused(h2s, bf, w1,
                               g1.reshape(1, H), b1.reshape(1, H),
                               w2, b2.reshape(1, D),
                               gnw.reshape(1, D), gnb.reshape(1, D),
                               gna.reshape(1, D))
        pools.append(pool)
    return jnp.concatenate(pools, axis=1)


# speedup vs baseline: 1.0123x; 1.0010x over previous
import jax
import jax.numpy as jnp
from jax import lax
from jax.experimental import pallas as pl
from jax.experimental.pallas import tpu as pltpu
from jax.experimental.pallas import tpu_sc as plsc

_F32 = jnp.float32
_BR = 512
_CH = 128
_NW = 32
_G = 64
_SELU_L = 1.0507009873554805
_SELU_A = 1.6732632423543772


def _selu(x):
    return _SELU_L * jnp.where(x > 0, x, _SELU_A * (jnp.exp(jnp.minimum(x, 0.0)) - 1.0))


def _make_sc_agg(NP, D, nchunks):
    mesh = plsc.VectorSubcoreMesh(core_axis_name="c", subcore_axis_name="s")
    rows_per = NP // 16
    Dh = D // 2

    def body(hlo_hbm, hhi_hbm, src_hbm, dst_hbm, out_hbm,
             idx_s, idx_d, r0, r1, r2, r3, acc_sh,
             sg0, sg1, sg2, sg3, ss0, ss1, ss2, ss3):
        rows = [r0, r1, r2, r3]
        sem_g = [sg0, sg1, sg2, sg3]
        sem_s = [ss0, ss1, ss2, ss3]
        c = lax.axis_index("c")
        s = lax.axis_index("s")
        pltpu.sync_copy(src_hbm.at[s], idx_s)
        pltpu.sync_copy(dst_hbm.at[s], idx_d)

        nbuf = len(rows)

        def pipeline(h_hbm):
            pltpu.sync_copy(h_hbm.at[pl.ds(s * rows_per, rows_per)],
                            acc_sh.at[pl.ds(s * rows_per, rows_per)])
            plsc.subcore_barrier()
            for b in range(nbuf):
                pltpu.async_copy(h_hbm.at[idx_s.at[b]], rows[b], sem_g[b])

            def step(k, carry):
                base = k * nbuf
                for b in range(nbuf):
                    ci = base + b
                    pltpu.make_async_copy(h_hbm.at[idx_s.at[ci]], rows[b],
                                          sem_g[b]).wait()
                    pltpu.async_copy(rows[b], acc_sh.at[idx_d.at[ci]],
                                     sem_s[b], add=True)
                for b in range(nbuf):
                    ci = base + nbuf + b

                    @pl.when(ci < nchunks)
                    def _(ci=ci, b=b):
                        pltpu.make_async_copy(rows[b],
                                              acc_sh.at[idx_d.at[base + b]],
                                              sem_s[b]).wait()
                        pltpu.async_copy(h_hbm.at[idx_s.at[ci]], rows[b],
                                         sem_g[b])
                return carry

            lax.fori_loop(0, nchunks // nbuf, step, 0)
            for b in range(nbuf):
                pltpu.make_async_copy(rows[b], acc_sh.at[idx_d.at[0]],
                                      sem_s[b]).wait()

        @pl.when(c == 0)
        def _():
            pipeline(hlo_hbm)

        @pl.when(c == 1)
        def _():
            pipeline(hhi_hbm)

        plsc.subcore_barrier()
        pltpu.sync_copy(acc_sh.at[pl.ds(s * rows_per, rows_per)],
                        out_hbm.at[c, pl.ds(s * rows_per, rows_per)])

    return pl.kernel(
        body,
        out_type=jax.ShapeDtypeStruct((2, NP, Dh), _F32),
        mesh=mesh,
        compiler_params=pltpu.CompilerParams(use_tc_tiling_on_sc=False),
        scratch_types=(
            [pltpu.VMEM((nchunks, _CH), jnp.int32),
             pltpu.VMEM((nchunks, _CH), jnp.int32)]
            + [pltpu.VMEM((_CH, Dh), _F32)] * 4
            + [pltpu.VMEM_SHARED((NP, Dh), _F32)]
            + [pltpu.SemaphoreType.DMA] * 8
        ),
    )


def _onehot(b_ref):
    bcol = b_ref[:, 0:1]
    gid = lax.broadcasted_iota(jnp.int32, (1, _G), 1).astype(_F32)
    return jnp.where(bcol == gid, 1.0, 0.0).astype(_F32)


def _make_fused(NP, D, H, nreal):
    NB = NP // _BR
    Dh = D // 2
    HIGH = lax.Precision.HIGHEST

    def body(a2_ref, b_ref, w1_ref, g1_ref, b1_ref, w2_ref, b2_ref,
             gnw_ref, gnb_ref, gna_ref, hlo_ref, hhi_ref, pool_ref,
             t_s, c_s, s_s, ab_s, stst_s, cnt8_s, mean_s, rstd_s, cntb_s):
        p = pl.program_id(0)
        i = pl.program_id(1)

        @pl.when(p == 0)
        def _():
            h2 = jnp.concatenate([a2_ref[0], a2_ref[1]], axis=1)
            rid = lax.broadcasted_iota(jnp.int32, (_BR, 1), 0) + i * _BR
            h2 = jnp.where(rid < nreal, h2, 0.0)
            h2c = h2.astype(jnp.bfloat16).astype(_F32)
            cb = lax.dot_general(h2c, h2c, (((0,), (0,)), ((), ())),
                                 preferred_element_type=_F32, precision=HIGH)
            sb = jnp.broadcast_to(jnp.sum(h2c, axis=0, keepdims=True), (8, D))

            @pl.when(i == 0)
            def _():
                c_s[...] = cb
                s_s[...] = sb

            @pl.when(i != 0)
            def _():
                c_s[...] += cb
                s_s[...] += sb

        @pl.when(p == 1)
        def _():
            @pl.when(i == 0)
            def _():
                w1 = w1_ref[...].astype(jnp.bfloat16).astype(_F32)
                mu = lax.dot_general(s_s[0:1, :], w1, (((1,), (1,)), ((), ())),
                                     preferred_element_type=_F32,
                                     precision=HIGH) / nreal
                wc = lax.dot_general(w1, c_s[...], (((1,), (0,)), ((), ())),
                                     preferred_element_type=_F32, precision=HIGH)
                m2col = jnp.sum(wc * w1, axis=1, keepdims=True) / nreal
                ii = lax.broadcasted_iota(jnp.int32, (H, H), 0)
                jj = lax.broadcasted_iota(jnp.int32, (H, H), 1)
                eye = jnp.where(ii == jj, 1.0, 0.0).astype(_F32)
                m2 = lax.dot_general(m2col, eye, (((0,), (0,)), ((), ())),
                                     preferred_element_type=_F32,
                                     precision=HIGH)
                rstd = lax.rsqrt(m2 - mu * mu + 1e-5)
                alpha = g1_ref[...] * rstd
                ab_s[0:1, :] = alpha
                ab_s[1:2, :] = b1_ref[...] - mu * alpha

            h2 = jnp.concatenate([a2_ref[0], a2_ref[1]], axis=1)
            y = lax.dot_general(h2.astype(jnp.bfloat16),
                                w1_ref[...].astype(jnp.bfloat16),
                                (((1,), (1,)), ((), ())),
                                preferred_element_type=_F32)
            z = _selu(y * ab_s[0:1, :] + ab_s[1:2, :])
            t = lax.dot_general(z.astype(jnp.bfloat16),
                                w2_ref[...].astype(jnp.bfloat16),
                                (((1,), (1,)), ((), ())),
                                preferred_element_type=_F32) + b2_ref[...]
            t_s[pl.ds(i * _BR, _BR), :] = t
            m = _onehot(b_ref)
            ts = jnp.concatenate([t, t * t], axis=1)
            stb = lax.dot_general(m, ts, (((0,), (0,)), ((), ())),
                                  preferred_element_type=_F32, precision=HIGH)
            cb = jnp.broadcast_to(jnp.sum(m, axis=0, keepdims=True), (8, _G))

            @pl.when(i == 0)
            def _():
                stst_s[...] = stb
                cnt8_s[...] = cb

            @pl.when(i != 0)
            def _():
                stst_s[...] += stb
                cnt8_s[...] += cb

        @pl.when(p == 2)
        def _():
            @pl.when(i == 0)
            def _():
                ii = lax.broadcasted_iota(jnp.int32, (_G, _G), 0)
                jj = lax.broadcasted_iota(jnp.int32, (_G, _G), 1)
                eye = jnp.where(ii == jj, 1.0, 0.0).astype(_F32)
                cntcol = lax.dot_general(eye, cnt8_s[0:1, :],
                                         (((1,), (1,)), ((), ())),
                                         preferred_element_type=_F32,
                                         precision=HIGH)
                cnt = jnp.maximum(cntcol, 1.0)
                mean = stst_s[:, :D] / cnt
                a = gna_ref[...]
                var = stst_s[:, D:] / cnt - (2.0 * a - a * a) * mean * mean
                mean_s[...] = mean
                rstd_s[...] = lax.rsqrt(var + 1e-5)
                cntb_s[...] = jnp.broadcast_to(cnt, (_G, D))
                pool_ref[...] = jnp.zeros((_G, D), _F32)

            t = t_s[pl.ds(i * _BR, _BR), :]
            m = _onehot(b_ref)
            meanb = lax.dot_general(m, mean_s[...], (((1,), (0,)), ((), ())),
                                    preferred_element_type=_F32, precision=HIGH)
            rstdb = lax.dot_general(m, rstd_s[...], (((1,), (0,)), ((), ())),
                                    preferred_element_type=_F32, precision=HIGH)
            out = (t - gna_ref[...] * meanb) * rstdb * gnw_ref[...] \
                + gnb_ref[...]
            hn = _selu(out)
            hlo_ref[...] = hn[:, :Dh]
            hhi_ref[...] = hn[:, Dh:]
            pool_ref[...] += lax.dot_general(m, hn, (((0,), (0,)), ((), ())),
                                             preferred_element_type=_F32,
                                             precision=HIGH) / cntb_s[...]

    return pl.pallas_call(
        body,
        grid=(3, NB),
        in_specs=[pl.BlockSpec((2, _BR, Dh), lambda p, i: (0, i * (p < 2), 0)),
                  pl.BlockSpec((_BR, D), lambda p, i: (i * (p > 0), 0)),
                  pl.BlockSpec((H, D), lambda p, i: (0, 0)),
                  pl.BlockSpec((1, H), lambda p, i: (0, 0)),
                  pl.BlockSpec((1, H), lambda p, i: (0, 0)),
                  pl.BlockSpec((D, H), lambda p, i: (0, 0)),
                  pl.BlockSpec((1, D), lambda p, i: (0, 0)),
                  pl.BlockSpec((1, D), lambda p, i: (0, 0)),
                  pl.BlockSpec((1, D), lambda p, i: (0, 0)),
                  pl.BlockSpec((1, D), lambda p, i: (0, 0))],
        out_specs=[pl.BlockSpec((_BR, Dh), lambda p, i: (i, 0)),
                   pl.BlockSpec((_BR, Dh), lambda p, i: (i, 0)),
                   pl.BlockSpec((_G, D), lambda p, i: (0, 0))],
        out_shape=[jax.ShapeDtypeStruct((NP, Dh), _F32),
                   jax.ShapeDtypeStruct((NP, Dh), _F32),
                   jax.ShapeDtypeStruct((_G, D), _F32)],
        scratch_shapes=[pltpu.VMEM((NP, D), _F32),
                        pltpu.VMEM((D, D), _F32),
                        pltpu.VMEM((8, D), _F32),
                        pltpu.VMEM((2, H), _F32),
                        pltpu.VMEM((_G, 2 * D), _F32),
                        pltpu.VMEM((8, _G), _F32),
                        pltpu.VMEM((_G, D), _F32),
                        pltpu.VMEM((_G, D), _F32),
                        pltpu.VMEM((_G, D), _F32)],
    )


def kernel(x, edge_index, batch,
           W1_1, g1_1, b1_1, W2_1, b2_1, gnw_1, gnb_1, gna_1,
           W1_2, g1_2, b1_2, W2_2, b2_2, gnw_2, gnb_2, gna_2,
           W1_3, g1_3, b1_3, W2_3, b2_3, gnw_3, gnb_3, gna_3,
           W1_4, g1_4, b1_4, W2_4, b2_4, gnw_4, gnb_4, gna_4):
    params = (W1_1, g1_1, b1_1, W2_1, b2_1, gnw_1, gnb_1, gna_1,
              W1_2, g1_2, b1_2, W2_2, b2_2, gnw_2, gnb_2, gna_2,
              W1_3, g1_3, b1_3, W2_3, b2_3, gnw_3, gnb_3, gna_3,
              W1_4, g1_4, b1_4, W2_4, b2_4, gnw_4, gnb_4, gna_4)
    N, D = x.shape
    E = edge_index.shape[1]
    H = W1_1.shape[0]
    Dh = D // 2
    NP = -(-N // _BR) * _BR
    nsub = 16
    nchunks = -(-E // (nsub * _CH))
    nchunks = -(-nchunks // 4) * 4
    EP = nsub * nchunks * _CH
    padr = NP - N

    pad_idx = (jnp.arange(EP - E, dtype=jnp.int32) % padr) + N
    src_r = jnp.concatenate([edge_index[0], pad_idx]).reshape(nsub, nchunks, _CH)
    dst_r = jnp.concatenate([edge_index[1], pad_idx]).reshape(nsub, nchunks, _CH)
    hlo = jnp.pad(x[:, :Dh], ((0, NP - N), (0, 0)))
    hhi = jnp.pad(x[:, Dh:], ((0, NP - N), (0, 0)))
    bp = jnp.pad(batch, (0, NP - N), constant_values=_G)
    bf = jnp.broadcast_to(bp.astype(_F32)[:, None], (NP, D))

    sc_agg = _make_sc_agg(NP, D, nchunks)
    fused = _make_fused(NP, D, H, N)

    pools = []
    for li in range(4):
        w1, g1, b1, w2, b2, gnw, gnb, gna = params[8 * li: 8 * li + 8]
        h2s = sc_agg(hlo, hhi, src_r, dst_r)
        hlo, hhi, pool = fused(h2s, bf, w1,
                               g1.reshape(1, H), b1.reshape(1, H),
                               w2, b2.reshape(1, D),
                               gnw.reshape(1, D), gnb.reshape(1, D),
                               gna.reshape(1, D))
        pools.append(pool)
    return jnp.concatenate(pools, axis=1)
